# Initial kernel scaffold; baseline (speedup 1.0000x reference)
import jax
import jax.numpy as jnp
from jax.experimental import pallas as pl


def kernel(X, edge_index, W1, b1, W2, b2, Wc, bc):
    raise NotImplementedError("write your pallas kernel here")


# speedup vs baseline: 42.8953x; 42.8953x over previous
import functools

import jax
import jax.numpy as jnp
from jax import lax
from jax.experimental import pallas as pl
from jax.experimental.pallas import tpu as pltpu
from jax.experimental.pallas import tpu_sc as plsc

N_NODES = 10000
NPAD = 10240
D = 64
NW = 32
CHUNK = 128
ROWS_PT = NPAD // 16

_mesh = functools.partial(
    plsc.VectorSubcoreMesh, core_axis_name="c", subcore_axis_name="s")


def _load_idx(real_hbm, pad_hbm, idx_v, wid, nch, nreal):
    npadr = 32 * nch - nreal
    nlast = nch - npadr
    if npadr == 0:
        pltpu.sync_copy(real_hbm.at[pl.ds(wid * nch, nch)], idx_v)
        return

    @pl.when(wid < NW - 1)
    def _full():
        pltpu.sync_copy(real_hbm.at[pl.ds(wid * nch, nch)], idx_v)

    @pl.when(wid == NW - 1)
    def _split():
        pltpu.sync_copy(real_hbm.at[pl.ds(nreal - nlast, nlast)],
                        idx_v.at[pl.ds(0, nlast)])
        pltpu.sync_copy(pad_hbm, idx_v.at[pl.ds(nlast, npadr)])


def _make_deg_kernel(nch, nreal):
    @functools.partial(
        pl.kernel,
        mesh=_mesh(),
        out_type=jax.ShapeDtypeStruct((2, NPAD), jnp.float32),
        scratch_types=[
            pltpu.VMEM((nch, CHUNK), jnp.int32),
            pltpu.VMEM((CHUNK,), jnp.float32),
            pltpu.VMEM_SHARED((NPAD,), jnp.float32),
            [pltpu.SemaphoreType.DMA] * 4,
        ],
    )
    def deg_kernel(dst_hbm, dpad_hbm, init_hbm, out_hbm,
                   dst_v, ones_v, acc_sh, ssems):
        c = lax.axis_index("c")
        s = lax.axis_index("s")
        wid = s * 2 + c
        _load_idx(dst_hbm, dpad_hbm, dst_v, wid, nch, nreal)
        pltpu.sync_copy(init_hbm.at[pl.ds(0, CHUNK)], ones_v)
        pltpu.sync_copy(init_hbm.at[pl.ds(s * ROWS_PT, ROWS_PT)],
                        acc_sh.at[pl.ds(s * ROWS_PT, ROWS_PT)])
        plsc.subcore_barrier()

        def body(i, _):
            for b in range(4):
                j = i * 4 + b

                @pl.when(j >= 4)
                def _drain():
                    pltpu.make_async_copy(
                        ones_v, acc_sh.at[dst_v.at[j - 4]], ssems[b]).wait()

                pltpu.async_copy(ones_v, acc_sh.at[dst_v.at[j]],
                                 ssems[b], add=True)
            return _

        lax.fori_loop(0, nch // 4, body, None)
        for b in range(4):
            pltpu.make_async_copy(ones_v,
                                  acc_sh.at[dst_v.at[nch - 4 + b]],
                                  ssems[b]).wait()
        plsc.subcore_barrier()
        pltpu.sync_copy(acc_sh.at[pl.ds(s * ROWS_PT, ROWS_PT)],
                        out_hbm.at[c, pl.ds(s * ROWS_PT, ROWS_PT)])

    return deg_kernel


def _make_agg_kernel(nch, nreal):
    kb = 2
    nbuf = 4
    nsup = nch // kb

    @functools.partial(
        pl.kernel,
        mesh=_mesh(),
        compiler_params=pltpu.CompilerParams(use_tc_tiling_on_sc=False),
        out_type=jax.ShapeDtypeStruct((2, NPAD, D), jnp.float32),
        scratch_types=[
            pltpu.VMEM((nch, CHUNK), jnp.int32),
            pltpu.VMEM((nch, CHUNK), jnp.int32),
            pltpu.VMEM((nbuf, kb, CHUNK, D), jnp.float32),
            pltpu.VMEM_SHARED((NPAD, D), jnp.float32),
            [pltpu.SemaphoreType.DMA] * 4,
            [pltpu.SemaphoreType.DMA] * 4,
        ],
    )
    def agg_kernel(y_hbm, src_hbm, spad_hbm, dst_hbm, dpad_hbm, out_hbm,
                   src_v, dst_v, rows_v, acc_sh, gsems, ssems):
        c = lax.axis_index("c")
        s = lax.axis_index("s")
        wid = s * 2 + c
        _load_idx(src_hbm, spad_hbm, src_v, wid, nch, nreal)
        _load_idx(dst_hbm, dpad_hbm, dst_v, wid, nch, nreal)
        pltpu.sync_copy(y_hbm.at[pl.ds(s * ROWS_PT, ROWS_PT)],
                        acc_sh.at[pl.ds(s * ROWS_PT, ROWS_PT)])
        plsc.subcore_barrier()

        def gather(g, b):
            for k in range(kb):
                pltpu.async_copy(y_hbm.at[src_v.at[g * kb + k]],
                                 rows_v.at[b, k], gsems[b])

        def gather_wait(g, b):
            for k in range(kb):
                pltpu.make_async_copy(y_hbm.at[src_v.at[g * kb + k]],
                                      rows_v.at[b, k], gsems[b]).wait()

        def scatter(g, b):
            for k in range(kb):
                pltpu.async_copy(rows_v.at[b, k],
                                 acc_sh.at[dst_v.at[g * kb + k]],
                                 ssems[b], add=True)

        def scatter_wait(g, b):
            for k in range(kb):
                pltpu.make_async_copy(rows_v.at[b, k],
                                      acc_sh.at[dst_v.at[g * kb + k]],
                                      ssems[b]).wait()

        gather(0, 0)
        gather(1, 1)

        def body(i, _):
            for bi in range(nbuf):
                g = i * nbuf + bi
                gather_wait(g, bi)
                scatter(g, bi)
                nb = (bi + 2) % nbuf

                @pl.when(g >= 2)
                def _drain():
                    scatter_wait(g - 2, nb)

                @pl.when(g + 2 < nsup)
                def _fire():
                    gather(g + 2, nb)
            return _

        lax.fori_loop(0, nsup // nbuf, body, None)
        scatter_wait(nsup - 2, (nsup - 2) % nbuf)
        scatter_wait(nsup - 1, (nsup - 1) % nbuf)
        plsc.subcore_barrier()
        pltpu.sync_copy(acc_sh.at[pl.ds(s * ROWS_PT, ROWS_PT)],
                        out_hbm.at[c, pl.ds(s * ROWS_PT, ROWS_PT)])

    return agg_kernel


_NB = 2048


def _tc_first(X, deg_pair, W1):
    def body(x_ref, d_ref, w_ref, y_ref):
        d = d_ref[0] + d_ref[1] - 1.0
        dis = lax.rsqrt(jnp.maximum(d, 1.0))
        y_ref[...] = dis * jnp.dot(x_ref[...], w_ref[...],
                                   preferred_element_type=jnp.float32)

    return pl.pallas_call(
        body,
        grid=(NPAD // _NB,),
        in_specs=[
            pl.BlockSpec((_NB, 128), lambda i: (i, 0)),
            pl.BlockSpec((2, _NB, 1), lambda i: (0, i, 0)),
            pl.BlockSpec((128, D), lambda i: (0, 0)),
        ],
        out_specs=pl.BlockSpec((_NB, D), lambda i: (i, 0)),
        out_shape=jax.ShapeDtypeStruct((NPAD, D), jnp.float32),
    )(X, deg_pair, W1)


def _tc_mid(S_pair, Y, deg_pair, W, b):
    def body(s_ref, y_ref, d_ref, w_ref, b_ref, o_ref):
        d = d_ref[0] + d_ref[1] - 1.0
        dis = lax.rsqrt(jnp.maximum(d, 1.0))
        c3 = dis * dis * dis
        tot = s_ref[0] + s_ref[1] - y_ref[...]
        h = jnp.maximum(tot * c3 + b_ref[...], 0.0)
        o_ref[...] = dis * jnp.dot(h, w_ref[...],
                                   preferred_element_type=jnp.float32)

    return pl.pallas_call(
        body,
        grid=(NPAD // _NB,),
        in_specs=[
            pl.BlockSpec((2, _NB, D), lambda i: (0, i, 0)),
            pl.BlockSpec((_NB, D), lambda i: (i, 0)),
            pl.BlockSpec((2, _NB, 1), lambda i: (0, i, 0)),
            pl.BlockSpec((D, D), lambda i: (0, 0)),
            pl.BlockSpec((1, D), lambda i: (0, 0)),
        ],
        out_specs=pl.BlockSpec((_NB, D), lambda i: (i, 0)),
        out_shape=jax.ShapeDtypeStruct((NPAD, D), jnp.float32),
    )(S_pair, Y, deg_pair, W, b)


def _tc_last(S_pair, Y, deg_pair, b2, Wc, bc):
    ncls = Wc.shape[1]
    nb = 2000

    def body(s_ref, y_ref, d_ref, b2_ref, w_ref, bc_ref, o_ref):
        d = d_ref[0] + d_ref[1] - 1.0
        dis = lax.rsqrt(jnp.maximum(d, 1.0))
        c3 = dis * dis * dis
        tot = s_ref[0] + s_ref[1] - y_ref[...]
        h = jnp.maximum(tot * c3 + b2_ref[...], 0.0)
        o_ref[...] = jnp.dot(h, w_ref[...],
                             preferred_element_type=jnp.float32) + bc_ref[...]

    return pl.pallas_call(
        body,
        grid=(N_NODES // nb,),
        in_specs=[
            pl.BlockSpec((2, nb, D), lambda i: (0, i, 0)),
            pl.BlockSpec((nb, D), lambda i: (i, 0)),
            pl.BlockSpec((2, nb, 1), lambda i: (0, i, 0)),
            pl.BlockSpec((1, D), lambda i: (0, 0)),
            pl.BlockSpec((D, ncls), lambda i: (0, 0)),
            pl.BlockSpec((1, ncls), lambda i: (0, 0)),
        ],
        out_specs=pl.BlockSpec((nb, ncls), lambda i: (i, 0)),
        out_shape=jax.ShapeDtypeStruct((N_NODES, ncls), jnp.float32),
    )(S_pair, Y, deg_pair, b2, Wc, bc)


def kernel(X, edge_index, W1, b1, W2, b2, Wc, bc):
    src = edge_index[0].astype(jnp.int32)
    dst = edge_index[1].astype(jnp.int32)
    n_edges = src.shape[0]
    if n_edges % CHUNK:
        extra = CHUNK - n_edges % CHUNK
        src = jnp.concatenate([src, jnp.zeros((extra,), jnp.int32)])
        dst = jnp.concatenate(
            [dst, jnp.full((extra,), N_NODES, jnp.int32)])
        n_edges += extra
    nreal = n_edges // CHUNK
    nch = -(-nreal // NW)
    nch = -(-nch // 8) * 8
    npadr = NW * nch - nreal

    src2d = src.reshape(nreal, CHUNK)
    dst2d = dst.reshape(nreal, CHUNK)
    rng = jnp.arange(npadr * CHUNK, dtype=jnp.int32)
    spad = (rng % N_NODES).reshape(npadr, CHUNK)
    dpad = (N_NODES + rng % (NPAD - N_NODES)).reshape(npadr, CHUNK)
    ones_init = jnp.ones((NPAD,), jnp.float32)

    deg_pair = _make_deg_kernel(nch, nreal)(dst2d, dpad, ones_init)
    degp = deg_pair.reshape(2, NPAD, 1)

    agg = _make_agg_kernel(nch, nreal)

    Y1 = _tc_first(X, degp, W1)
    S1 = agg(Y1, src2d, spad, dst2d, dpad)
    Y2 = _tc_mid(S1, Y1, degp, W2, b1.reshape(1, -1))
    S2 = agg(Y2, src2d, spad, dst2d, dpad)
    return _tc_last(S2, Y2, degp, b2.reshape(1, -1),
                    Wc, bc.reshape(1, -1))
